# baseline (speedup 1.0000x reference)
import functools

import jax
import jax.numpy as jnp
import numpy as np
from jax import lax
from jax.experimental import pallas as pl
from jax.experimental.pallas import tpu as pltpu
from jax.experimental.pallas import tpu_sc as plsc

B, C, S = 32, 32, 8192
BMIN, BMAX = 30, 50
START_IDX, END_IDX = 4500, 5250
MASK_RATIO = 5 * 0.5 / 9.0

LANES = 16
IDX_CHUNK = 128


def _gen_blocks(rng, available_indices, total_mask_length):
    min_size, max_size = BMIN, BMAX
    mask_positions = []
    remaining = total_mask_length
    arr = np.array(available_indices)
    rng.shuffle(arr)
    available_indices = arr.tolist()
    while remaining >= min_size and available_indices:
        block_size = min(
            max_size,
            remaining,
            int(rng.integers(min_size, min(max_size, remaining) + 1)),
        )
        a = np.asarray(available_indices)
        n = len(a) - block_size + 1
        if n <= 0:
            valid_starts = []
        else:
            ok = np.ones(n, dtype=bool)
            base = a[:n]
            for j in range(1, block_size):
                ok &= a[j : j + n] == base + j
            valid_starts = np.nonzero(ok)[0].tolist()
        if not valid_starts:
            positions = available_indices[:remaining]
            mask_positions.extend(positions[:block_size])
            remaining -= len(positions[:block_size])
            break
        start_idx = valid_starts[int(rng.integers(len(valid_starts)))]
        block_positions = available_indices[start_idx : start_idx + block_size]
        mask_positions.extend(block_positions)
        remaining -= block_size
        for pos in block_positions:
            available_indices.remove(pos)
    return sorted(set(mask_positions))


def _build_index_tables():
    rng = np.random.default_rng(0)
    available = list(range(0, START_IDX)) + list(range(END_IDX, S))
    total_mask_length = int(len(available) * MASK_RATIO)
    iota = np.arange(S)
    p_rows, g_rows, k_max = [], [], 0
    for _ in range(B):
        src = np.arange(S)
        if total_mask_length >= BMIN and rng.random() < 1.0:
            for pos in _gen_blocks(rng, list(available), total_mask_length):
                if pos > 0:
                    src[pos] = src[pos - 1]
                elif pos < S - 1:
                    src[pos] = src[pos + 1]
        p = np.nonzero(src != iota)[0]
        g = src[p]
        assert np.all(src[g] == g)
        p_rows.append(p)
        g_rows.append(g)
        k_max = max(k_max, len(p))
    k_pad = max(LANES, -(-k_max // LANES) * LANES)
    pad = START_IDX
    p_tab = np.full((B, k_pad), pad, np.int32)
    g_tab = np.full((B, k_pad), pad, np.int32)
    for b in range(B):
        p_tab[b, : len(p_rows[b])] = p_rows[b]
        g_tab[b, : len(g_rows[b])] = g_rows[b]
    r_tab = np.repeat(np.arange(RCHUNK, dtype=np.int32), k_pad)[None, :].repeat(B, 0)
    p_loc = np.tile(p_tab, (1, RCHUNK))
    g_loc = np.tile(g_tab, (1, RCHUNK))
    tab = np.stack([p_loc, g_loc, r_tab], axis=1)
    return tab.astype(np.int32), RCHUNK * k_pad


RCHUNK = 1
NCHUNKS = C // RCHUNK
NBUF = 10
KAHEAD = 5

_TAB, _K_LOC = _build_index_tables()


def _sc_body(x_hbm, tab_hbm, out_hbm, tabv, bufs, in_sems, out_sems):
    b = lax.axis_index("s") * 2 + lax.axis_index("c")
    pltpu.sync_copy(tab_hbm.at[b], tabv)
    pv, gv, rv = tabv.at[0], tabv.at[1], tabv.at[2]

    def src_at(i):
        return x_hbm.at[b, pl.ds(i * RCHUNK, RCHUNK)]

    def dst_at(i):
        return out_hbm.at[b, pl.ds(i * RCHUNK, RCHUNK)]

    def issue_in(i, bi):
        pltpu.async_copy(src_at(i), bufs.at[bi], in_sems.at[bi])

    def prime(i, _):
        issue_in(i, i)
        return 0

    lax.fori_loop(0, KAHEAD, prime, 0)

    def step(i, _):
        bi = lax.rem(i, NBUF)
        pltpu.make_async_copy(src_at(i), bufs.at[bi], in_sems.at[bi]).wait()
        bvec = jnp.full((LANES,), bi, jnp.int32)

        def fix(j, _):
            r = rv[pl.ds(j * LANES, LANES)]
            g = gv[pl.ds(j * LANES, LANES)]
            p = pv[pl.ds(j * LANES, LANES)]
            vals = plsc.load_gather(bufs, [bvec, r, g])
            plsc.store_scatter(bufs, [bvec, r, p], vals)
            return 0

        lax.fori_loop(0, _K_LOC // LANES, fix, 0)
        pltpu.async_copy(bufs.at[bi], dst_at(i), out_sems.at[bi])
        nxt = i + KAHEAD

        @pl.when(nxt < NCHUNKS)
        def _():
            nbi = lax.rem(nxt, NBUF)
            prev = nxt - NBUF

            @pl.when(prev >= 0)
            def _():
                pltpu.make_async_copy(
                    bufs.at[nbi], dst_at(prev), out_sems.at[nbi]
                ).wait()

            pltpu.async_copy(src_at(nxt), bufs.at[nbi], in_sems.at[nbi])

        return 0

    lax.fori_loop(0, NCHUNKS, step, 0)

    def drain(i, _):
        bi = lax.rem(i, NBUF)
        pltpu.make_async_copy(bufs.at[bi], dst_at(i), out_sems.at[bi]).wait()
        return 0

    lax.fori_loop(max(0, NCHUNKS - NBUF), NCHUNKS, drain, 0)


def kernel(x):
    tab = jnp.asarray(_TAB)
    mesh = plsc.VectorSubcoreMesh(core_axis_name="c", subcore_axis_name="s")
    run = functools.partial(
        pl.kernel,
        mesh=mesh,
        out_type=jax.ShapeDtypeStruct((B, C, S), jnp.float32),
        scratch_types=[
            pltpu.VMEM((3, _K_LOC), jnp.int32),
            pltpu.VMEM((NBUF, RCHUNK, S), jnp.float32),
            pltpu.SemaphoreType.DMA((NBUF,)),
            pltpu.SemaphoreType.DMA((NBUF,)),
        ],
        compiler_params=pltpu.CompilerParams(
            needs_layout_passes=False, skip_device_barrier=True
        ),
    )(_sc_body)
    return run(x, tab)


# speedup vs baseline: 1.0175x; 1.0175x over previous
import functools

import jax
import jax.numpy as jnp
import numpy as np
from jax import lax
from jax.experimental import pallas as pl
from jax.experimental.pallas import tpu as pltpu
from jax.experimental.pallas import tpu_sc as plsc

B, C, S = 32, 32, 8192
BMIN, BMAX = 30, 50
START_IDX, END_IDX = 4500, 5250
MASK_RATIO = 5 * 0.5 / 9.0

LANES = 16


def _gen_blocks(rng, available_indices, total_mask_length):
    min_size, max_size = BMIN, BMAX
    mask_positions = []
    remaining = total_mask_length
    arr = np.array(available_indices)
    rng.shuffle(arr)
    available_indices = arr.tolist()
    while remaining >= min_size and available_indices:
        block_size = min(
            max_size,
            remaining,
            int(rng.integers(min_size, min(max_size, remaining) + 1)),
        )
        a = np.asarray(available_indices)
        n = len(a) - block_size + 1
        if n <= 0:
            valid_starts = []
        else:
            ok = np.ones(n, dtype=bool)
            base = a[:n]
            for j in range(1, block_size):
                ok &= a[j : j + n] == base + j
            valid_starts = np.nonzero(ok)[0].tolist()
        if not valid_starts:
            positions = available_indices[:remaining]
            mask_positions.extend(positions[:block_size])
            remaining -= len(positions[:block_size])
            break
        start_idx = valid_starts[int(rng.integers(len(valid_starts)))]
        block_positions = available_indices[start_idx : start_idx + block_size]
        mask_positions.extend(block_positions)
        remaining -= block_size
        for pos in block_positions:
            available_indices.remove(pos)
    return sorted(set(mask_positions))


def _build_index_tables():
    rng = np.random.default_rng(0)
    available = list(range(0, START_IDX)) + list(range(END_IDX, S))
    total_mask_length = int(len(available) * MASK_RATIO)
    iota = np.arange(S)
    p_rows, g_rows, k_max = [], [], 0
    for _ in range(B):
        src = np.arange(S)
        if total_mask_length >= BMIN and rng.random() < 1.0:
            for pos in _gen_blocks(rng, list(available), total_mask_length):
                if pos > 0:
                    src[pos] = src[pos - 1]
                elif pos < S - 1:
                    src[pos] = src[pos + 1]
        p = np.nonzero(src != iota)[0]
        g = src[p]
        assert np.all(src[g] == g)
        p_rows.append(p)
        g_rows.append(g)
        k_max = max(k_max, len(p))
    k_pad = max(LANES, -(-k_max // LANES) * LANES)
    pad = START_IDX
    p_tab = np.full((B, k_pad), pad, np.int32)
    g_tab = np.full((B, k_pad), pad, np.int32)
    for b in range(B):
        p_tab[b, : len(p_rows[b])] = p_rows[b]
        g_tab[b, : len(g_rows[b])] = g_rows[b]
    r_tab = np.repeat(np.arange(RCHUNK, dtype=np.int32), k_pad)[None, :].repeat(B, 0)
    p_loc = np.tile(p_tab, (1, RCHUNK))
    g_loc = np.tile(g_tab, (1, RCHUNK))
    tab = np.stack([p_loc, g_loc, r_tab], axis=1)
    return tab.astype(np.int32), RCHUNK * k_pad


RCHUNK = 1
NCHUNKS = C // RCHUNK
NBUF = 12
KAHEAD = 6

_TAB, _K_LOC = _build_index_tables()


def _sc_body(x_hbm, tab_hbm, out_hbm, tabv, bufs, in_sems, out_sems):
    b = lax.axis_index("s") * 2 + lax.axis_index("c")
    pltpu.sync_copy(tab_hbm.at[b], tabv)
    pv, gv, rv = tabv.at[0], tabv.at[1], tabv.at[2]

    def src_at(i):
        return x_hbm.at[b, pl.ds(i * RCHUNK, RCHUNK)]

    def dst_at(i):
        return out_hbm.at[b, pl.ds(i * RCHUNK, RCHUNK)]

    def issue_in(i, bi):
        pltpu.async_copy(src_at(i), bufs.at[bi], in_sems.at[bi])

    def prime(i, _):
        issue_in(i, i)
        return 0

    lax.fori_loop(0, KAHEAD, prime, 0)

    def step(i, _):
        bi = lax.rem(i, NBUF)
        pltpu.make_async_copy(src_at(i), bufs.at[bi], in_sems.at[bi]).wait()
        bvec = jnp.full((LANES,), bi, jnp.int32)

        def fix(j, _):
            r = rv[pl.ds(j * LANES, LANES)]
            g = gv[pl.ds(j * LANES, LANES)]
            p = pv[pl.ds(j * LANES, LANES)]
            vals = plsc.load_gather(bufs, [bvec, r, g])
            plsc.store_scatter(bufs, [bvec, r, p], vals)
            return 0

        lax.fori_loop(0, _K_LOC // LANES, fix, 0)
        pltpu.async_copy(bufs.at[bi], dst_at(i), out_sems.at[bi])
        nxt = i + KAHEAD

        @pl.when(nxt < NCHUNKS)
        def _():
            nbi = lax.rem(nxt, NBUF)
            prev = nxt - NBUF

            @pl.when(prev >= 0)
            def _():
                pltpu.make_async_copy(
                    bufs.at[nbi], dst_at(prev), out_sems.at[nbi]
                ).wait()

            pltpu.async_copy(src_at(nxt), bufs.at[nbi], in_sems.at[nbi])

        return 0

    lax.fori_loop(0, NCHUNKS, step, 0)

    def drain(i, _):
        bi = lax.rem(i, NBUF)
        pltpu.make_async_copy(bufs.at[bi], dst_at(i), out_sems.at[bi]).wait()
        return 0

    lax.fori_loop(max(0, NCHUNKS - NBUF), NCHUNKS, drain, 0)


def kernel(x):
    tab = jnp.asarray(_TAB)
    mesh = plsc.VectorSubcoreMesh(core_axis_name="c", subcore_axis_name="s")
    run = functools.partial(
        pl.kernel,
        mesh=mesh,
        out_type=jax.ShapeDtypeStruct((B, C, S), jnp.float32),
        scratch_types=[
            pltpu.VMEM((3, _K_LOC), jnp.int32),
            pltpu.VMEM((NBUF, RCHUNK, S), jnp.float32),
            pltpu.SemaphoreType.DMA((NBUF,)),
            pltpu.SemaphoreType.DMA((NBUF,)),
        ],
        compiler_params=pltpu.CompilerParams(
            needs_layout_passes=False, skip_device_barrier=True
        ),
    )(_sc_body)
    return run(x, tab)


# speedup vs baseline: 1.0262x; 1.0085x over previous
import functools

import jax
import jax.numpy as jnp
import numpy as np
from jax import lax
from jax.experimental import pallas as pl
from jax.experimental.pallas import tpu as pltpu
from jax.experimental.pallas import tpu_sc as plsc

B, C, S = 32, 32, 8192
BMIN, BMAX = 30, 50
START_IDX, END_IDX = 4500, 5250
MASK_RATIO = 5 * 0.5 / 9.0

LANES = 16


def _gen_blocks(rng, available_indices, total_mask_length):
    min_size, max_size = BMIN, BMAX
    mask_positions = []
    remaining = total_mask_length
    arr = np.array(available_indices)
    rng.shuffle(arr)
    available_indices = arr.tolist()
    while remaining >= min_size and available_indices:
        block_size = min(
            max_size,
            remaining,
            int(rng.integers(min_size, min(max_size, remaining) + 1)),
        )
        a = np.asarray(available_indices)
        n = len(a) - block_size + 1
        if n <= 0:
            valid_starts = []
        else:
            ok = np.ones(n, dtype=bool)
            base = a[:n]
            for j in range(1, block_size):
                ok &= a[j : j + n] == base + j
            valid_starts = np.nonzero(ok)[0].tolist()
        if not valid_starts:
            positions = available_indices[:remaining]
            mask_positions.extend(positions[:block_size])
            remaining -= len(positions[:block_size])
            break
        start_idx = valid_starts[int(rng.integers(len(valid_starts)))]
        block_positions = available_indices[start_idx : start_idx + block_size]
        mask_positions.extend(block_positions)
        remaining -= block_size
        for pos in block_positions:
            available_indices.remove(pos)
    return sorted(set(mask_positions))


def _build_index_tables():
    rng = np.random.default_rng(0)
    available = list(range(0, START_IDX)) + list(range(END_IDX, S))
    total_mask_length = int(len(available) * MASK_RATIO)
    iota = np.arange(S)
    p_rows, g_rows, k_max = [], [], 0
    for _ in range(B):
        src = np.arange(S)
        if total_mask_length >= BMIN and rng.random() < 1.0:
            for pos in _gen_blocks(rng, list(available), total_mask_length):
                if pos > 0:
                    src[pos] = src[pos - 1]
                elif pos < S - 1:
                    src[pos] = src[pos + 1]
        p = np.nonzero(src != iota)[0]
        g = src[p]
        assert np.all(src[g] == g)
        p_rows.append(p)
        g_rows.append(g)
        k_max = max(k_max, len(p))
    k_pad = max(LANES, -(-k_max // LANES) * LANES)
    pad = START_IDX
    p_tab = np.full((B, k_pad), pad, np.int32)
    g_tab = np.full((B, k_pad), pad, np.int32)
    for b in range(B):
        p_tab[b, : len(p_rows[b])] = p_rows[b]
        g_tab[b, : len(g_rows[b])] = g_rows[b]
    r_tab = np.repeat(np.arange(RCHUNK, dtype=np.int32), k_pad)[None, :].repeat(B, 0)
    p_loc = np.tile(p_tab, (1, RCHUNK))
    g_loc = np.tile(g_tab, (1, RCHUNK))
    tab = np.stack([p_loc, g_loc, r_tab], axis=1)
    return tab.astype(np.int32), RCHUNK * k_pad


RCHUNK = 1
NCHUNKS = C // RCHUNK
NBUF = 12
KAHEAD = 6

_TAB, _K_LOC = _build_index_tables()


def _sc_body(x_hbm, tab_hbm, out_hbm, tabv, bufs, in_sems, out_sems):
    b = lax.axis_index("s") * 2 + lax.axis_index("c")
    pv, gv, rv = tabv.at[0], tabv.at[1], tabv.at[2]

    def src_at(i):
        return x_hbm.at[b, pl.ds(i * RCHUNK, RCHUNK)]

    def dst_at(i):
        return out_hbm.at[b, pl.ds(i * RCHUNK, RCHUNK)]

    def issue_in(i, bi):
        pltpu.async_copy(src_at(i), bufs.at[bi], in_sems.at[bi])

    def prime(i, _):
        issue_in(i, i)
        return 0

    lax.fori_loop(0, KAHEAD, prime, 0)
    pltpu.sync_copy(tab_hbm.at[b], tabv)

    def step(i, _):
        bi = lax.rem(i, NBUF)
        pltpu.make_async_copy(src_at(i), bufs.at[bi], in_sems.at[bi]).wait()
        bvec = jnp.full((LANES,), bi, jnp.int32)

        def fix(j, _):
            r = rv[pl.ds(j * LANES, LANES)]
            g = gv[pl.ds(j * LANES, LANES)]
            p = pv[pl.ds(j * LANES, LANES)]
            vals = plsc.load_gather(bufs, [bvec, r, g])
            plsc.store_scatter(bufs, [bvec, r, p], vals)
            return 0

        lax.fori_loop(0, _K_LOC // LANES, fix, 0)
        pltpu.async_copy(bufs.at[bi], dst_at(i), out_sems.at[bi])
        nxt = i + KAHEAD

        @pl.when(nxt < NCHUNKS)
        def _():
            nbi = lax.rem(nxt, NBUF)
            prev = nxt - NBUF

            @pl.when(prev >= 0)
            def _():
                pltpu.make_async_copy(
                    bufs.at[nbi], dst_at(prev), out_sems.at[nbi]
                ).wait()

            pltpu.async_copy(src_at(nxt), bufs.at[nbi], in_sems.at[nbi])

        return 0

    lax.fori_loop(0, NCHUNKS, step, 0)

    def drain(i, _):
        bi = lax.rem(i, NBUF)
        pltpu.make_async_copy(bufs.at[bi], dst_at(i), out_sems.at[bi]).wait()
        return 0

    lax.fori_loop(max(0, NCHUNKS - NBUF), NCHUNKS, drain, 0)


def kernel(x):
    tab = jnp.asarray(_TAB)
    mesh = plsc.VectorSubcoreMesh(core_axis_name="c", subcore_axis_name="s")
    run = functools.partial(
        pl.kernel,
        mesh=mesh,
        out_type=jax.ShapeDtypeStruct((B, C, S), jnp.float32),
        scratch_types=[
            pltpu.VMEM((3, _K_LOC), jnp.int32),
            pltpu.VMEM((NBUF, RCHUNK, S), jnp.float32),
            pltpu.SemaphoreType.DMA((NBUF,)),
            pltpu.SemaphoreType.DMA((NBUF,)),
        ],
        compiler_params=pltpu.CompilerParams(
            needs_layout_passes=False, skip_device_barrier=True
        ),
    )(_sc_body)
    return run(x, tab)
